# baseline (speedup 1.0000x reference)
import functools

import jax
import jax.numpy as jnp
from jax import lax
from jax.experimental import pallas as pl
from jax.experimental.pallas import tpu as pltpu
from jax.experimental.pallas import tpu_sc as plsc


_B, _Q, _N = 4, 32, 1024
_K = 512
_NW = 32
_PTS = (_B * _N) // _NW
_NV = _PTS // 16


def _sc_body(ze_hbm, emb_hbm, out_hbm, z_vm, e_vm, res_vm):
    c = lax.axis_index("c")
    s = lax.axis_index("s")
    w = s * 2 + c
    b = w // (_N // _PTS)
    n0 = (w % (_N // _PTS)) * _PTS
    pltpu.sync_copy(emb_hbm, e_vm)
    pltpu.sync_copy(ze_hbm.at[b, :, pl.ds(n0, _PTS)], z_vm)

    big = jnp.full((16,), 3.0e38, jnp.float32)

    def k_step(k, mins):
        rows = [e_vm[k, pl.ds(0, 16)], e_vm[k, pl.ds(16, 16)]]
        accs = [jnp.zeros((16,), jnp.float32) for _ in range(_NV)]
        for q in range(_Q):
            ekq = rows[q // 16][q % 16]
            for nv in range(_NV):
                d = z_vm[q, pl.ds(nv * 16, 16)] - ekq
                accs[nv] = accs[nv] + d * d
        return tuple(jnp.minimum(m, a) for m, a in zip(mins, accs))

    mins = lax.fori_loop(0, _K, k_step, tuple(big for _ in range(_NV)))
    for nv in range(_NV):
        res_vm[pl.ds(nv * 16, 16)] = mins[nv]
    pltpu.sync_copy(res_vm, out_hbm.at[b, pl.ds(n0, _PTS)])


def kernel(ze, emb):
    B, Q, N = ze.shape
    K, _ = emb.shape
    run = functools.partial(
        pl.kernel,
        mesh=plsc.VectorSubcoreMesh(core_axis_name="c", subcore_axis_name="s"),
        out_type=jax.ShapeDtypeStruct((B, N), jnp.float32),
        scratch_types=[
            pltpu.VMEM((Q, _PTS), jnp.float32),
            pltpu.VMEM((K, Q), jnp.float32),
            pltpu.VMEM((_PTS,), jnp.float32),
        ],
    )(_sc_body)
    return run(ze, emb)


# speedup vs baseline: 26.6269x; 26.6269x over previous
import jax
import jax.numpy as jnp
from jax.experimental import pallas as pl


def _l2_min_body(ze_ref, emb_ref, out_ref):
    e = emb_ref[...]
    en = e * -2.0
    ee = jnp.sum(e * e, axis=1, keepdims=True)
    B = ze_ref.shape[0]
    for b in range(B):
        z = ze_ref[b]
        dot = jax.lax.dot_general(
            en, z, (((1,), (0,)), ((), ())),
            preferred_element_type=jnp.float32,
            precision=jax.lax.Precision.DEFAULT,
        )
        zz = jnp.sum(z * z, axis=0)
        out_ref[b, :] = jnp.min(dot + ee, axis=0) + zz


def kernel(ze, emb):
    B, Q, N = ze.shape
    K, _ = emb.shape
    return pl.pallas_call(
        _l2_min_body,
        out_shape=jax.ShapeDtypeStruct((B, N), jnp.float32),
    )(ze, emb)
